# baseline (speedup 1.0000x reference)
import functools

import jax
import jax.numpy as jnp
from jax import lax
from jax.experimental import pallas as pl
from jax.experimental.pallas import tpu as pltpu
from jax.experimental.pallas import tpu_sc as plsc

B, S = 4, 4096
DIM = 128
N_TOTAL = B * S

_info = plsc.get_sparse_core_info()
_NC, _NS = _info.num_cores, _info.num_subcores
_NW = _NC * _NS
_PER_W = N_TOTAL // _NW


_CHUNK = 64
_NCH = _PER_W // _CHUNK


def _emb_kernel(idx_hbm, tok_hbm, segtab_hbm, out_hbm,
                idx_v, rows_v, seg_spmem, sem_g, sem_a, sem_s):
    sid = lax.axis_index("s")
    wid = sid * _NC + lax.axis_index("c")
    base = wid * _PER_W
    pltpu.sync_copy(idx_hbm.at[wid], idx_v)

    gt = [None] * _NCH
    ad = [None] * _NCH
    so = [None] * _NCH
    for c in range(_NCH):
        gt[c] = pltpu.async_copy(
            tok_hbm.at[idx_v.at[0, c]], rows_v.at[c], sem_g.at[c])

    @pl.when(sid == 0)
    def _stage():
        pltpu.sync_copy(segtab_hbm, seg_spmem)
    plsc.subcore_barrier()

    def store_out(c):
        return pltpu.async_copy(
            rows_v.at[c], out_hbm.at[pl.ds(base + c * _CHUNK, _CHUNK)],
            sem_s.at[c])

    for c in range(_NCH):
        gt[c].wait()
        ad[c] = pltpu.async_copy(
            seg_spmem.at[idx_v.at[1, c]], rows_v.at[c], sem_a.at[c],
            add=True)
        if c >= 1:
            ad[c - 1].wait()
            so[c - 1] = store_out(c - 1)
    ad[_NCH - 1].wait()
    so[_NCH - 1] = store_out(_NCH - 1)
    for c in range(_NCH):
        so[c].wait()


@jax.jit
def _embeddings(idx_packed, tok_table, seg_table):
    mesh = plsc.VectorSubcoreMesh(core_axis_name="c", subcore_axis_name="s")
    return pl.kernel(
        _emb_kernel,
        out_type=jax.ShapeDtypeStruct((N_TOTAL, DIM), jnp.float32),
        mesh=mesh,
        scratch_types=[
            pltpu.VMEM((2, _NCH, _CHUNK), jnp.int32),
            pltpu.VMEM((_NCH, _CHUNK, DIM), jnp.float32),
            pltpu.VMEM_SHARED((512, DIM), jnp.float32),
            pltpu.SemaphoreType.DMA((_NCH,)),
            pltpu.SemaphoreType.DMA((_NCH,)),
            pltpu.SemaphoreType.DMA((_NCH,)),
        ],
    )(idx_packed, tok_table, seg_table)


def kernel(x, seg, tok_table, seg_table):
    idx_packed = jnp.stack(
        [x.reshape(_NW, _NCH, _CHUNK), seg.reshape(_NW, _NCH, _CHUNK)],
        axis=1)
    out = _embeddings(idx_packed, tok_table, seg_table)
    return out.reshape(B, S, DIM)


# speedup vs baseline: 1.0081x; 1.0081x over previous
import jax
import jax.numpy as jnp
from jax import lax
from jax.experimental import pallas as pl
from jax.experimental.pallas import tpu as pltpu
from jax.experimental.pallas import tpu_sc as plsc

B, S = 4, 4096
DIM = 128
N_TOTAL = B * S

_info = plsc.get_sparse_core_info()
_NC, _NS = _info.num_cores, _info.num_subcores
_NW = _NC * _NS
_PER_W = N_TOTAL // _NW


_CHUNK = 64
_NCH = _PER_W // _CHUNK


def _emb_kernel(idx_hbm, tok_hbm, segtab_hbm, out_hbm,
                idx_v, rows_v, seg_spmem, sem_g, sem_a, sem_s):
    sid = lax.axis_index("s")
    wid = sid * _NC + lax.axis_index("c")
    base = wid * _PER_W
    pltpu.sync_copy(idx_hbm.at[wid], idx_v)

    gt = [None] * _NCH
    ad = [None] * _NCH
    so = [None] * _NCH
    for c in range(_NCH):
        gt[c] = pltpu.async_copy(
            tok_hbm.at[idx_v.at[0, c]], rows_v.at[c], sem_g.at[c])

    @pl.when(sid == 0)
    def _stage():
        pltpu.sync_copy(segtab_hbm, seg_spmem)
    plsc.subcore_barrier()

    for c in range(_NCH):
        gt[c].wait()
        ad[c] = pltpu.async_copy(
            seg_spmem.at[idx_v.at[1, c]], rows_v.at[c], sem_a.at[c],
            add=True)
    for c in range(_NCH):
        ad[c].wait()
        so[c] = pltpu.async_copy(
            rows_v.at[c], out_hbm.at[pl.ds(base + c * _CHUNK, _CHUNK)],
            sem_s.at[c])
    for c in range(_NCH):
        so[c].wait()


@jax.jit
def _embeddings(idx_packed, tok_table, seg_table):
    mesh = plsc.VectorSubcoreMesh(core_axis_name="c", subcore_axis_name="s")
    return pl.kernel(
        _emb_kernel,
        out_type=jax.ShapeDtypeStruct((N_TOTAL, DIM), jnp.float32),
        mesh=mesh,
        scratch_types=[
            pltpu.VMEM((2, _NCH, _CHUNK), jnp.int32),
            pltpu.VMEM((_NCH, _CHUNK, DIM), jnp.float32),
            pltpu.VMEM_SHARED((512, DIM), jnp.float32),
            pltpu.SemaphoreType.DMA((_NCH,)),
            pltpu.SemaphoreType.DMA((_NCH,)),
            pltpu.SemaphoreType.DMA((_NCH,)),
        ],
    )(idx_packed, tok_table, seg_table)


def kernel(x, seg, tok_table, seg_table):
    idx_packed = jnp.stack(
        [x.reshape(_NW, _NCH, _CHUNK), seg.reshape(_NW, _NCH, _CHUNK)],
        axis=1)
    out = _embeddings(idx_packed, tok_table, seg_table)
    return out.reshape(B, S, DIM)
